# baseline (speedup 1.0000x reference)
import jax
import jax.numpy as jnp
from jax.experimental import pallas as pl
from jax.experimental.pallas import tpu as pltpu

_RANKS = (8, 16, 32, 8, 16, 32, 8, 16)
_OFFS = tuple(sum(_RANKS[:a]) for a in range(len(_RANKS)))
_ALPHA = 1.0
_RMAX = 32
_NUM_ADAPTERS = 8
_SBLK = 2048
_NCHUNK = 16
_CROWS = _SBLK // _NCHUNK

_NT = (((1,), (1,)), ((), ()))
_NN = (((1,), (0,)), ((), ()))


def _lora_kernel(x_ref, bcat_ref, atcat_ref, o_ref, bs_ref, as_ref, sem):
    step = pl.program_id(0)

    @pl.when(step == 0)
    def _prep():
        copies = []
        for a in range(_NUM_ADAPTERS):
            r, off = _RANKS[a], _OFFS[a]
            copies.append(pltpu.make_async_copy(
                bcat_ref.at[off:off + r, :], bs_ref.at[a, :r, :], sem))
            copies.append(pltpu.make_async_copy(
                atcat_ref.at[off:off + r, :], as_ref.at[a, :r, :], sem))
        for c in copies:
            c.start()
        for a in range(_NUM_ADAPTERS):
            r = _RANKS[a]
            if r < _RMAX:
                bs_ref[a, r:, :] = jnp.zeros((_RMAX - r, bs_ref.shape[2]), jnp.float32)
                as_ref[a, r:, :] = jnp.zeros((_RMAX - r, as_ref.shape[2]), jnp.float32)
        for c in copies:
            c.wait()
        for a in range(_NUM_ADAPTERS):
            r = _RANKS[a]
            bs_ref[a, :r, :] = bs_ref[a, :r, :] * (_ALPHA / r)

    aid = step % _NUM_ADAPTERS
    bsel = bs_ref[aid]
    asel = as_ref[aid]
    for k in range(_NCHUNK):
        xb = x_ref[pl.ds(k * _CROWS, _CROWS), :]
        y = jax.lax.dot_general(xb, bsel, _NT,
                                preferred_element_type=jnp.float32)
        o_ref[pl.ds(k * _CROWS, _CROWS), :] = jax.lax.dot_general(
            y, asel, _NN, preferred_element_type=jnp.float32)


def kernel(x, A0, B0, A1, B1, A2, B2, A3, B3, A4, B4, A5, B5, A6, B6, A7, B7):
    As = (A0, A1, A2, A3, A4, A5, A6, A7)
    Bs = (B0, B1, B2, B3, B4, B5, B6, B7)
    B, S, D = x.shape
    out_f = A0.shape[0]

    bcat = jnp.concatenate(Bs, axis=0)
    atcat = jnp.concatenate([a.T for a in As], axis=0)

    x2 = x.reshape(B * S, D)
    out2 = pl.pallas_call(
        _lora_kernel,
        grid=(B,),
        in_specs=[
            pl.BlockSpec((_SBLK, D), lambda b: (b, 0)),
            pl.BlockSpec(memory_space=pl.ANY),
            pl.BlockSpec(memory_space=pl.ANY),
        ],
        out_specs=pl.BlockSpec((_SBLK, D), lambda b: (b, 0)),
        out_shape=jax.ShapeDtypeStruct((B * S, out_f), x.dtype),
        scratch_shapes=[
            pltpu.VMEM((_NUM_ADAPTERS, _RMAX, D), jnp.float32),
            pltpu.VMEM((_NUM_ADAPTERS, _RMAX, out_f), jnp.float32),
            pltpu.SemaphoreType.DMA,
        ],
    )(x2, bcat, atcat)
    return out2.reshape(B, S, out_f)


# speedup vs baseline: 1.0021x; 1.0021x over previous
import jax
import jax.numpy as jnp
from jax.experimental import pallas as pl
from jax.experimental.pallas import tpu as pltpu

_RANKS = (8, 16, 32, 8, 16, 32, 8, 16)
_ALPHA = 1.0
_RMAX = 32
_NUM_ADAPTERS = 8
_SBLK = 2048
_NCHUNK = 16
_CROWS = _SBLK // _NCHUNK

_NT = (((1,), (1,)), ((), ()))


def _lora_kernel(x_ref, *refs):
    w_refs = refs[:16]
    o_ref = refs[16]
    bs_ref = refs[17]
    as_ref = refs[18]
    step = pl.program_id(0)

    @pl.when(step == 0)
    def _prep():
        bs_ref[...] = jnp.zeros_like(bs_ref)
        as_ref[...] = jnp.zeros_like(as_ref)
        for a in range(_NUM_ADAPTERS):
            r = _RANKS[a]
            a_w = w_refs[2 * a][...]
            b_w = w_refs[2 * a + 1][...]
            bs_ref[a, :r, :] = b_w
            as_ref[a, :, :r] = a_w * (_ALPHA / r)

    aid = step % _NUM_ADAPTERS
    bsel = bs_ref[aid]
    asel = as_ref[aid]
    for k in range(_NCHUNK):
        xb = x_ref[pl.ds(k * _CROWS, _CROWS), :]
        y = jax.lax.dot_general(xb, bsel, _NT,
                                preferred_element_type=jnp.float32)
        o_ref[pl.ds(k * _CROWS, _CROWS), :] = jax.lax.dot_general(
            y, asel, _NT, preferred_element_type=jnp.float32)


def kernel(x, A0, B0, A1, B1, A2, B2, A3, B3, A4, B4, A5, B5, A6, B6, A7, B7):
    ws = (A0, B0, A1, B1, A2, B2, A3, B3, A4, B4, A5, B5, A6, B6, A7, B7)
    B, S, D = x.shape
    out_f = A0.shape[0]

    x2 = x.reshape(B * S, D)
    w_specs = [pl.BlockSpec(w.shape, lambda b: (0, 0)) for w in ws]
    out2 = pl.pallas_call(
        _lora_kernel,
        grid=(B,),
        in_specs=[pl.BlockSpec((_SBLK, D), lambda b: (b, 0))] + w_specs,
        out_specs=pl.BlockSpec((_SBLK, D), lambda b: (b, 0)),
        out_shape=jax.ShapeDtypeStruct((B * S, out_f), x.dtype),
        scratch_shapes=[
            pltpu.VMEM((_NUM_ADAPTERS, _RMAX, D), jnp.float32),
            pltpu.VMEM((_NUM_ADAPTERS, out_f, _RMAX), jnp.float32),
        ],
    )(x2, *ws)
    return out2.reshape(B, S, out_f)
